# baseline (speedup 1.0000x reference)
import functools

import jax
import jax.numpy as jnp
import numpy as np
from jax import lax
from jax.experimental import pallas as pl
from jax.experimental.pallas import tpu as pltpu
from jax.experimental.pallas import tpu_sc as plsc

_V_MIN = -10.0
_V_MAX = 10.0
_NUM_ATOMS = 51
_DISCOUNT = 0.99
_ATOM_DELTA = (_V_MAX - _V_MIN) / (_NUM_ATOMS - 1)
_ATOMS_F32 = np.asarray(
    [_V_MIN + _ATOM_DELTA * i for i in range(_NUM_ATOMS)], dtype=np.float32
)

_NC = 2
_NS = 16
_L = 16
_NW = _NC * _NS


@functools.lru_cache(maxsize=None)
def _make_kernel(bs: int, num_atoms: int):
    A = num_atoms
    rows_per_w = bs // _NW
    chunk = min(256, rows_per_w)
    n_chunks = rows_per_w // chunk
    groups = chunk // _L

    mesh = plsc.VectorSubcoreMesh(
        core_axis_name="c", subcore_axis_name="s",
        num_cores=_NC, num_subcores=_NS,
    )

    @functools.partial(
        pl.kernel,
        out_type=jax.ShapeDtypeStruct((bs, A), jnp.float32),
        mesh=mesh,
        compiler_params=pltpu.CompilerParams(
            needs_layout_passes=False, use_tc_tiling_on_sc=True),
        scratch_types=[
            pltpu.VMEM((rows_per_w,), jnp.float32),
            pltpu.VMEM((rows_per_w,), jnp.float32),
            pltpu.VMEM((chunk, A), jnp.float32),
            pltpu.VMEM((chunk, A), jnp.float32),
        ],
    )
    def projection_kernel(rew_hbm, nd_hbm, probs_hbm, out_hbm,
                          rew_v, nd_v, probs_v, acc_v):
        cid = lax.axis_index("c")
        sid = lax.axis_index("s")
        wid = sid * _NC + cid
        row0 = wid * rows_per_w
        pltpu.sync_copy(rew_hbm.at[pl.ds(row0, rows_per_w)], rew_v)
        pltpu.sync_copy(nd_hbm.at[pl.ds(row0, rows_per_w)], nd_v)
        lanes = lax.iota(jnp.int32, _L)
        zeros = jnp.zeros((_L,), jnp.float32)

        def chunk_body(t, carry):
            cbase = t * chunk
            pltpu.sync_copy(probs_hbm.at[pl.ds(row0 + cbase, chunk)], probs_v)

            def group_body(g, gcarry):
                rbase = g * _L
                r = rew_v[pl.ds(cbase + rbase, _L)]
                nd = nd_v[pl.ds(cbase + rbase, _L)]
                c = _DISCOUNT * nd
                rows16 = rbase + lanes
                for kk in range(A):
                    plsc.store_scatter(
                        acc_v, [rows16, jnp.full((_L,), kk, jnp.int32)], zeros)
                for j in range(A):
                    pj = plsc.load_gather(
                        probs_v, [rows16, jnp.full((_L,), j, jnp.int32)])
                    z = r + c * float(_ATOMS_F32[j])
                    z = jnp.maximum(z, _V_MIN)
                    z = jnp.minimum(z, _V_MAX)
                    b = (z - _V_MIN) / _ATOM_DELTA
                    li = b.astype(jnp.int32)
                    frac = b - li.astype(jnp.float32)
                    wl = (1.0 - frac) * pj
                    wu = frac * pj
                    iu = jnp.minimum(li + 1, A - 1)
                    plsc.addupdate_scatter(acc_v, [rows16, li], wl)
                    plsc.addupdate_scatter(acc_v, [rows16, iu], wu)
                return gcarry

            lax.fori_loop(0, groups, group_body, 0)
            pltpu.sync_copy(acc_v, out_hbm.at[pl.ds(row0 + cbase, chunk)])
            return carry

        lax.fori_loop(0, n_chunks, chunk_body, 0)

    return projection_kernel


def kernel(reward, probs, not_done):
    bs, A = probs.shape
    run = _make_kernel(bs, A)
    return run(reward.reshape(bs), not_done.reshape(bs), probs)


# speedup vs baseline: 1.2072x; 1.2072x over previous
import functools

import jax
import jax.numpy as jnp
import numpy as np
from jax import lax
from jax.experimental import pallas as pl
from jax.experimental.pallas import tpu as pltpu
from jax.experimental.pallas import tpu_sc as plsc

_V_MIN = -10.0
_V_MAX = 10.0
_NUM_ATOMS = 51
_DISCOUNT = 0.99
_ATOM_DELTA = (_V_MAX - _V_MIN) / (_NUM_ATOMS - 1)
_ATOMS_F32 = np.asarray(
    [_V_MIN + _ATOM_DELTA * i for i in range(_NUM_ATOMS)], dtype=np.float32
)

_NC = 2
_NS = 16
_L = 16
_NW = _NC * _NS


@functools.lru_cache(maxsize=None)
def _make_kernel(bs: int, num_atoms: int):
    A = num_atoms
    rows_per_w = bs // _NW
    chunk = min(256, rows_per_w)
    n_chunks = rows_per_w // chunk
    groups = chunk // _L

    mesh = plsc.VectorSubcoreMesh(
        core_axis_name="c", subcore_axis_name="s",
        num_cores=_NC, num_subcores=_NS,
    )

    @functools.partial(
        pl.kernel,
        out_type=jax.ShapeDtypeStruct((bs, A), jnp.float32),
        mesh=mesh,
        compiler_params=pltpu.CompilerParams(
            needs_layout_passes=False, use_tc_tiling_on_sc=True),
        scratch_types=[
            pltpu.VMEM((rows_per_w,), jnp.float32),
            pltpu.VMEM((rows_per_w,), jnp.float32),
            pltpu.VMEM((chunk, _NUM_ATOMS), jnp.float32),
            pltpu.VMEM((chunk * _NUM_ATOMS + _L,), jnp.float32),
            pltpu.VMEM((chunk * _NUM_ATOMS + _L,), jnp.float32),
            pltpu.VMEM((chunk, _NUM_ATOMS), jnp.float32),
        ],
    )
    def projection_kernel(rew_hbm, nd_hbm, probs_hbm, out_hbm,
                          rew_v, nd_v, stage_in, probs_l, acc_l, stage_out):
        cid = lax.axis_index("c")
        sid = lax.axis_index("s")
        wid = sid * _NC + cid
        row0 = wid * rows_per_w
        pltpu.sync_copy(rew_hbm.at[pl.ds(row0, rows_per_w)], rew_v)
        pltpu.sync_copy(nd_hbm.at[pl.ds(row0, rows_per_w)], nd_v)
        lanes = lax.iota(jnp.int32, _L)
        zeros = jnp.zeros((_L,), jnp.float32)
        tailmask = lanes < (A - 48)

        def chunk_body(t, carry):
            cbase = t * chunk
            pltpu.sync_copy(probs_hbm.at[pl.ds(row0 + cbase, chunk)], stage_in)

            def repack_in(rr, c2):
                for u in range(8):
                    r = rr * 8 + u
                    base = r * A
                    probs_l[pl.ds(base, _L)] = stage_in[r, pl.ds(0, _L)]
                    probs_l[pl.ds(base + _L, _L)] = stage_in[r, pl.ds(_L, _L)]
                    probs_l[pl.ds(base + 2 * _L, _L)] = (
                        stage_in[r, pl.ds(2 * _L, _L)])
                    probs_l[pl.ds(base + A - _L, _L)] = (
                        stage_in[r, pl.ds(A - _L, _L)])
                return c2

            lax.fori_loop(0, chunk // 8, repack_in, 0)

            def group_body(g, gcarry):
                rbase = g * _L
                r = rew_v[pl.ds(cbase + rbase, _L)]
                nd = nd_v[pl.ds(cbase + rbase, _L)]
                c = _DISCOUNT * nd
                rowb = (rbase + lanes) * A
                rowmax = rowb + (A - 1)
                gb = rbase * A
                for kk in range(A):
                    acc_l[pl.ds(gb + kk * _L, _L)] = zeros
                for j in range(A):
                    pj = plsc.load_gather(probs_l, [rowb + j])
                    z = r + c * float(_ATOMS_F32[j])
                    z = jnp.maximum(z, _V_MIN)
                    z = jnp.minimum(z, _V_MAX)
                    b = (z - _V_MIN) / _ATOM_DELTA
                    li = b.astype(jnp.int32)
                    frac = b - li.astype(jnp.float32)
                    wl = (1.0 - frac) * pj
                    wu = frac * pj
                    idxl = rowb + li
                    idxu = jnp.minimum(idxl + 1, rowmax)
                    plsc.addupdate_scatter(acc_l, [idxl], wl)
                    plsc.addupdate_scatter(acc_l, [idxu], wu)
                return gcarry

            lax.fori_loop(0, groups, group_body, 0)

            def repack_out(rr, c3):
                for u in range(8):
                    r = rr * 8 + u
                    base = r * A
                    stage_out[r, pl.ds(0, _L)] = acc_l[pl.ds(base, _L)]
                    stage_out[r, pl.ds(_L, _L)] = acc_l[pl.ds(base + _L, _L)]
                    stage_out[r, pl.ds(2 * _L, _L)] = (
                        acc_l[pl.ds(base + 2 * _L, _L)])
                    stage_out[r, pl.ds(A - _L, _L)] = (
                        acc_l[pl.ds(base + A - _L, _L)])
                return c3

            lax.fori_loop(0, chunk // 8, repack_out, 0)
            pltpu.sync_copy(stage_out, out_hbm.at[pl.ds(row0 + cbase, chunk)])
            return carry

        lax.fori_loop(0, n_chunks, chunk_body, 0)

    return projection_kernel


def kernel(reward, probs, not_done):
    bs, A = probs.shape
    run = _make_kernel(bs, A)
    return run(reward.reshape(bs), not_done.reshape(bs), probs)


# speedup vs baseline: 1.5748x; 1.3044x over previous
import functools

import jax
import jax.numpy as jnp
import numpy as np
from jax import lax
from jax.experimental import pallas as pl
from jax.experimental.pallas import tpu as pltpu
from jax.experimental.pallas import tpu_sc as plsc

_V_MIN = -10.0
_V_MAX = 10.0
_NUM_ATOMS = 51
_DISCOUNT = 0.99
_ATOM_DELTA = (_V_MAX - _V_MIN) / (_NUM_ATOMS - 1)
_ATOMS_F32 = np.asarray(
    [_V_MIN + _ATOM_DELTA * i for i in range(_NUM_ATOMS)], dtype=np.float32
)

_NC = 2
_NS = 16
_L = 16
_NW = _NC * _NS


@functools.lru_cache(maxsize=None)
def _make_kernel(bs: int, num_atoms: int):
    A = num_atoms
    rows_per_w = bs // _NW
    chunk = min(256, rows_per_w)
    n_chunks = rows_per_w // chunk
    groups = chunk // _L

    mesh = plsc.VectorSubcoreMesh(
        core_axis_name="c", subcore_axis_name="s",
        num_cores=_NC, num_subcores=_NS,
    )

    @functools.partial(
        pl.kernel,
        out_type=jax.ShapeDtypeStruct((bs, A), jnp.float32),
        mesh=mesh,
        compiler_params=pltpu.CompilerParams(
            needs_layout_passes=False, use_tc_tiling_on_sc=True),
        scratch_types=[
            pltpu.VMEM((rows_per_w,), jnp.float32),
            pltpu.VMEM((rows_per_w,), jnp.float32),
            pltpu.VMEM((chunk, _NUM_ATOMS), jnp.float32),
            pltpu.VMEM((chunk * _NUM_ATOMS,), jnp.float32),
            pltpu.VMEM((chunk * _NUM_ATOMS,), jnp.float32),
            pltpu.VMEM((chunk, _NUM_ATOMS), jnp.float32),
        ],
    )
    def projection_kernel(rew_hbm, nd_hbm, probs_hbm, out_hbm,
                          rew_v, nd_v, stage_in, probs_l, acc_l, stage_out):
        cid = lax.axis_index("c")
        sid = lax.axis_index("s")
        wid = sid * _NC + cid
        row0 = wid * rows_per_w
        pltpu.sync_copy(rew_hbm.at[pl.ds(row0, rows_per_w)], rew_v)
        pltpu.sync_copy(nd_hbm.at[pl.ds(row0, rows_per_w)], nd_v)
        lanes = lax.iota(jnp.int32, _L)
        zeros = jnp.zeros((_L,), jnp.float32)

        def chunk_body(t, carry):
            cbase = t * chunk
            pltpu.sync_copy(probs_hbm.at[pl.ds(row0 + cbase, chunk)], stage_in)

            @plsc.parallel_loop(0, chunk, unroll=4)
            def repack_in(r):
                base = r * A
                a0 = stage_in[r, pl.ds(0, _L)]
                a1 = stage_in[r, pl.ds(_L, _L)]
                a2 = stage_in[r, pl.ds(2 * _L, _L)]
                a3 = stage_in[r, pl.ds(A - _L, _L)]
                probs_l[pl.ds(base, _L)] = a0
                probs_l[pl.ds(base + _L, _L)] = a1
                probs_l[pl.ds(base + 2 * _L, _L)] = a2
                probs_l[pl.ds(base + A - _L, _L)] = a3

            @plsc.parallel_loop(0, groups)
            def group_body(g):
                rbase = g * _L
                r = rew_v[pl.ds(cbase + rbase, _L)]
                nd = nd_v[pl.ds(cbase + rbase, _L)]
                c = _DISCOUNT * nd
                rowb = (rbase + lanes) * A
                rowmax = rowb + (A - 1)
                gb = rbase * A
                for kk in range(A):
                    acc_l[pl.ds(gb + kk * _L, _L)] = zeros
                for j in range(A):
                    pj = plsc.load_gather(probs_l, [rowb + j])
                    z = r + c * float(_ATOMS_F32[j])
                    z = jnp.maximum(z, _V_MIN)
                    z = jnp.minimum(z, _V_MAX)
                    b = (z - _V_MIN) / _ATOM_DELTA
                    li = b.astype(jnp.int32)
                    frac = b - li.astype(jnp.float32)
                    wl = (1.0 - frac) * pj
                    wu = frac * pj
                    idxl = rowb + li
                    idxu = jnp.minimum(idxl + 1, rowmax)
                    plsc.addupdate_scatter(acc_l, [idxl], wl)
                    plsc.addupdate_scatter(acc_l, [idxu], wu)

            @plsc.parallel_loop(0, chunk, unroll=4)
            def repack_out(r):
                base = r * A
                b0 = acc_l[pl.ds(base, _L)]
                b1 = acc_l[pl.ds(base + _L, _L)]
                b2 = acc_l[pl.ds(base + 2 * _L, _L)]
                b3 = acc_l[pl.ds(base + A - _L, _L)]
                stage_out[r, pl.ds(0, _L)] = b0
                stage_out[r, pl.ds(_L, _L)] = b1
                stage_out[r, pl.ds(2 * _L, _L)] = b2
                stage_out[r, pl.ds(A - _L, _L)] = b3

            pltpu.sync_copy(stage_out, out_hbm.at[pl.ds(row0 + cbase, chunk)])
            return carry

        lax.fori_loop(0, n_chunks, chunk_body, 0)

    return projection_kernel


def kernel(reward, probs, not_done):
    bs, A = probs.shape
    run = _make_kernel(bs, A)
    return run(reward.reshape(bs), not_done.reshape(bs), probs)


# speedup vs baseline: 1.8630x; 1.1830x over previous
import functools

import jax
import jax.numpy as jnp
import numpy as np
from jax import lax
from jax.experimental import pallas as pl
from jax.experimental.pallas import tpu as pltpu
from jax.experimental.pallas import tpu_sc as plsc

_V_MIN = -10.0
_V_MAX = 10.0
_NUM_ATOMS = 51
_DISCOUNT = 0.99
_ATOM_DELTA = (_V_MAX - _V_MIN) / (_NUM_ATOMS - 1)
_ATOMS_F32 = np.asarray(
    [_V_MIN + _ATOM_DELTA * i for i in range(_NUM_ATOMS)], dtype=np.float32
)

_NC = 2
_NS = 16
_L = 16
_NW = _NC * _NS


@functools.lru_cache(maxsize=None)
def _make_kernel(bs: int, num_atoms: int):
    A = num_atoms
    rows_per_w = bs // _NW
    chunk = min(128, rows_per_w)
    n_chunks = rows_per_w // chunk
    groups = chunk // _L
    assert n_chunks % 2 == 0

    mesh = plsc.VectorSubcoreMesh(
        core_axis_name="c", subcore_axis_name="s",
        num_cores=_NC, num_subcores=_NS,
    )

    @functools.partial(
        pl.kernel,
        out_type=jax.ShapeDtypeStruct((bs, A), jnp.float32),
        mesh=mesh,
        compiler_params=pltpu.CompilerParams(
            needs_layout_passes=False, use_tc_tiling_on_sc=True),
        scratch_types=[
            pltpu.VMEM((rows_per_w,), jnp.float32),
            pltpu.VMEM((rows_per_w,), jnp.float32),
            pltpu.VMEM((chunk, _NUM_ATOMS), jnp.float32),
            pltpu.VMEM((chunk, _NUM_ATOMS), jnp.float32),
            pltpu.VMEM((chunk, _NUM_ATOMS), jnp.float32),
            pltpu.VMEM((chunk, _NUM_ATOMS), jnp.float32),
            pltpu.VMEM((chunk * _NUM_ATOMS,), jnp.float32),
            pltpu.VMEM((chunk * _NUM_ATOMS,), jnp.float32),
            pltpu.SemaphoreType.DMA,
            pltpu.SemaphoreType.DMA,
            pltpu.SemaphoreType.DMA,
            pltpu.SemaphoreType.DMA,
        ],
    )
    def projection_kernel(rew_hbm, nd_hbm, probs_hbm, out_hbm,
                          rew_v, nd_v, si0, si1, so0, so1,
                          probs_l, acc_l, isem0, isem1, osem0, osem1):
        cid = lax.axis_index("c")
        sid = lax.axis_index("s")
        wid = sid * _NC + cid
        row0 = wid * rows_per_w
        pltpu.sync_copy(rew_hbm.at[pl.ds(row0, rows_per_w)], rew_v)
        pltpu.sync_copy(nd_hbm.at[pl.ds(row0, rows_per_w)], nd_v)
        lanes = lax.iota(jnp.int32, _L)
        zeros = jnp.zeros((_L,), jnp.float32)
        stages_in = (si0, si1)
        stages_out = (so0, so1)
        isems = (isem0, isem1)
        osems = (osem0, osem1)

        def in_rows(t):
            return probs_hbm.at[pl.ds(row0 + t * chunk, chunk)]

        def out_rows(t):
            return out_hbm.at[pl.ds(row0 + t * chunk, chunk)]

        pltpu.async_copy(in_rows(0), si0, isem0)
        pltpu.async_copy(in_rows(1), si1, isem1)

        def super_body(tt, carry):
            for b in range(2):
                t = tt * 2 + b
                s_in = stages_in[b]
                s_out = stages_out[b]
                pltpu.make_async_copy(in_rows(t), s_in, isems[b]).wait()

                @plsc.parallel_loop(0, chunk, unroll=4)
                def repack_in(r):
                    base = r * A
                    a0 = s_in[r, pl.ds(0, _L)]
                    a1 = s_in[r, pl.ds(_L, _L)]
                    a2 = s_in[r, pl.ds(2 * _L, _L)]
                    a3 = s_in[r, pl.ds(A - _L, _L)]
                    probs_l[pl.ds(base, _L)] = a0
                    probs_l[pl.ds(base + _L, _L)] = a1
                    probs_l[pl.ds(base + 2 * _L, _L)] = a2
                    probs_l[pl.ds(base + A - _L, _L)] = a3

                @pl.when(t + 2 < n_chunks)
                def _():
                    pltpu.async_copy(in_rows(t + 2), s_in, isems[b])

                cbase = t * chunk

                @plsc.parallel_loop(0, groups)
                def group_body(g):
                    rbase = g * _L
                    r = rew_v[pl.ds(cbase + rbase, _L)]
                    nd = nd_v[pl.ds(cbase + rbase, _L)]
                    c = _DISCOUNT * nd
                    rowb = (rbase + lanes) * A
                    rowmax = rowb + (A - 1)
                    gb = rbase * A
                    for kk in range(A):
                        acc_l[pl.ds(gb + kk * _L, _L)] = zeros
                    for j in range(A):
                        pj = plsc.load_gather(probs_l, [rowb + j])
                        z = r + c * float(_ATOMS_F32[j])
                        z = jnp.maximum(z, _V_MIN)
                        z = jnp.minimum(z, _V_MAX)
                        bb = (z - _V_MIN) / _ATOM_DELTA
                        li = bb.astype(jnp.int32)
                        frac = bb - li.astype(jnp.float32)
                        wl = (1.0 - frac) * pj
                        wu = frac * pj
                        idxl = rowb + li
                        idxu = jnp.minimum(idxl + 1, rowmax)
                        plsc.addupdate_scatter(acc_l, [idxl], wl)
                        plsc.addupdate_scatter(acc_l, [idxu], wu)

                @pl.when(t >= 2)
                def _():
                    pltpu.make_async_copy(s_out, out_rows(t - 2),
                                          osems[b]).wait()

                @plsc.parallel_loop(0, chunk, unroll=4)
                def repack_out(r):
                    base = r * A
                    b0 = acc_l[pl.ds(base, _L)]
                    b1 = acc_l[pl.ds(base + _L, _L)]
                    b2 = acc_l[pl.ds(base + 2 * _L, _L)]
                    b3 = acc_l[pl.ds(base + A - _L, _L)]
                    s_out[r, pl.ds(0, _L)] = b0
                    s_out[r, pl.ds(_L, _L)] = b1
                    s_out[r, pl.ds(2 * _L, _L)] = b2
                    s_out[r, pl.ds(A - _L, _L)] = b3

                pltpu.async_copy(s_out, out_rows(t), osems[b])
            return carry

        lax.fori_loop(0, n_chunks // 2, super_body, 0)
        pltpu.make_async_copy(so0, out_rows(n_chunks - 2), osem0).wait()
        pltpu.make_async_copy(so1, out_rows(n_chunks - 1), osem1).wait()

    return projection_kernel


def kernel(reward, probs, not_done):
    bs, A = probs.shape
    run = _make_kernel(bs, A)
    return run(reward.reshape(bs), not_done.reshape(bs), probs)


# speedup vs baseline: 3.4487x; 1.8512x over previous
import functools

import jax
import jax.numpy as jnp
import numpy as np
from jax import lax
from jax.experimental import pallas as pl
from jax.experimental.pallas import tpu as pltpu
from jax.experimental.pallas import tpu_sc as plsc

_V_MIN = -10.0
_V_MAX = 10.0
_NUM_ATOMS = 51
_DISCOUNT = 0.99
_ATOM_DELTA = (_V_MAX - _V_MIN) / (_NUM_ATOMS - 1)
_ATOMS_F32 = np.asarray(
    [_V_MIN + _ATOM_DELTA * i for i in range(_NUM_ATOMS)], dtype=np.float32
)

_NC = 2
_NS = 16
_L = 16
_NW = _NC * _NS


@functools.lru_cache(maxsize=None)
def _make_kernel(bs: int, num_atoms: int):
    A = num_atoms
    cols_per_w = bs // _NW
    chunk = min(256, cols_per_w)
    n_chunks = cols_per_w // chunk
    groups = chunk // _L
    assert n_chunks % 2 == 0

    mesh = plsc.VectorSubcoreMesh(
        core_axis_name="c", subcore_axis_name="s",
        num_cores=_NC, num_subcores=_NS,
    )

    @functools.partial(
        pl.kernel,
        out_type=jax.ShapeDtypeStruct((A, bs), jnp.float32),
        mesh=mesh,
        compiler_params=pltpu.CompilerParams(
            needs_layout_passes=False, use_tc_tiling_on_sc=True),
        scratch_types=[
            pltpu.VMEM((cols_per_w,), jnp.float32),
            pltpu.VMEM((cols_per_w,), jnp.float32),
            pltpu.VMEM((A, chunk), jnp.float32),
            pltpu.VMEM((A, chunk), jnp.float32),
            pltpu.VMEM((A, chunk), jnp.float32),
            pltpu.VMEM((A, chunk), jnp.float32),
            pltpu.VMEM((A * chunk,), jnp.float32),
            pltpu.SemaphoreType.DMA,
            pltpu.SemaphoreType.DMA,
            pltpu.SemaphoreType.DMA,
            pltpu.SemaphoreType.DMA,
        ],
    )
    def projection_kernel(rew_hbm, nd_hbm, probs_hbm, out_hbm,
                          rew_v, nd_v, si0, si1, so0, so1,
                          acc_l, isem0, isem1, osem0, osem1):
        cid = lax.axis_index("c")
        sid = lax.axis_index("s")
        wid = sid * _NC + cid
        col0 = wid * cols_per_w
        pltpu.sync_copy(rew_hbm.at[pl.ds(col0, cols_per_w)], rew_v)
        pltpu.sync_copy(nd_hbm.at[pl.ds(col0, cols_per_w)], nd_v)
        lanes = lax.iota(jnp.int32, _L)
        zeros = jnp.zeros((_L,), jnp.float32)
        stages_in = (si0, si1)
        stages_out = (so0, so1)
        isems = (isem0, isem1)
        osems = (osem0, osem1)

        def in_cols(t):
            return probs_hbm.at[:, pl.ds(col0 + t * chunk, chunk)]

        def out_cols(t):
            return out_hbm.at[:, pl.ds(col0 + t * chunk, chunk)]

        pltpu.async_copy(in_cols(0), si0, isem0)
        pltpu.async_copy(in_cols(1), si1, isem1)

        def super_body(tt, carry):
            for bsel in range(2):
                t = tt * 2 + bsel
                s_in = stages_in[bsel]
                s_out = stages_out[bsel]
                pltpu.make_async_copy(in_cols(t), s_in, isems[bsel]).wait()
                cbase = t * chunk

                @plsc.parallel_loop(0, groups)
                def group_body(g):
                    c0 = g * _L
                    r = rew_v[pl.ds(cbase + c0, _L)]
                    nd = nd_v[pl.ds(cbase + c0, _L)]
                    c = _DISCOUNT * nd
                    colv = c0 + lanes
                    maxidx = (A - 1) * chunk + colv
                    for kk in range(A):
                        acc_l[pl.ds(kk * chunk + c0, _L)] = zeros
                    for j in range(A):
                        pj = s_in[j, pl.ds(c0, _L)]
                        z = r + c * float(_ATOMS_F32[j])
                        z = jnp.maximum(z, _V_MIN)
                        z = jnp.minimum(z, _V_MAX)
                        bb = (z - _V_MIN) / _ATOM_DELTA
                        li = bb.astype(jnp.int32)
                        frac = bb - li.astype(jnp.float32)
                        wl = (1.0 - frac) * pj
                        wu = frac * pj
                        idxl = li * chunk + colv
                        idxu = jnp.minimum(idxl + chunk, maxidx)
                        plsc.addupdate_scatter(acc_l, [idxl], wl)
                        plsc.addupdate_scatter(acc_l, [idxu], wu)

                @pl.when(t + 2 < n_chunks)
                def _():
                    pltpu.async_copy(in_cols(t + 2), s_in, isems[bsel])

                @pl.when(t >= 2)
                def _():
                    pltpu.make_async_copy(s_out, out_cols(t - 2),
                                          osems[bsel]).wait()

                @plsc.parallel_loop(0, A, unroll=2)
                def repack_out(a):
                    base = a * chunk
                    for cc in range(groups):
                        s_out[a, pl.ds(cc * _L, _L)] = (
                            acc_l[pl.ds(base + cc * _L, _L)])

                pltpu.async_copy(s_out, out_cols(t), osems[bsel])
            return carry

        lax.fori_loop(0, n_chunks // 2, super_body, 0)
        pltpu.make_async_copy(so0, out_cols(n_chunks - 2), osem0).wait()
        pltpu.make_async_copy(so1, out_cols(n_chunks - 1), osem1).wait()

    return projection_kernel


def kernel(reward, probs, not_done):
    bs, A = probs.shape
    run = _make_kernel(bs, A)
    out_t = run(reward.reshape(bs), not_done.reshape(bs), probs.T)
    return out_t.T
